# baseline (speedup 1.0000x reference)
import functools

import jax
import jax.numpy as jnp
import numpy as np
from jax import lax
from jax.experimental import pallas as pl
from jax.experimental.pallas import tpu as pltpu
from jax.experimental.pallas import tpu_sc as plsc

N_POS = 262144
N_LEVELS = 16
N_FEAT = 2
LOG2_T = 19
TABLE_SIZE = 1 << LOG2_T
BASE_RES = 16
PER_LEVEL_SCALE = 1.3819129
P1 = np.uint32(2654435761)
P2 = np.uint32(805459861)

NW = 32
PW = N_POS // NW
CH = 1024
NCHUNK = PW // CH
NGROUP = CH // 16
NPAIR = CH // 8
NSUB = 2 * CH // 128
OB = CH * 32

RES_LIST = [int(np.floor(BASE_RES * (PER_LEVEL_SCALE ** l))) for l in range(N_LEVELS)]

RES0 = RES_LIST[0]
S0 = RES0 + 2
E0 = S0 * S0 * S0
E0P = ((2 * E0 + 2047) // 2048) * 1024
NLUTG = E0P // 16
TSHARE = 2 * E0P // 16
NLUTPT = TSHARE // 16
NLUTST = TSHARE // 128
OFF0 = [bx * S0 * S0 + by * S0 + bz
        for bz, by, bx in [((c >> 2) & 1, (c >> 1) & 1, c & 1) for c in range(8)]]


def _sc_body(xs_hbm, ys_hbm, zs_hbm, tbl_hbm, w_hbm, res_hbm, out_hbm,
             xs_v, ys_v, zs_v, idx2_v, wgt_v, rows_v, out_v, w_v, res_v,
             lut0_v, lut1_v, lut_sh, sem, psem, osem):
    sid = lax.axis_index("s")
    wid = sid * 2 + lax.axis_index("c")
    lanes = lax.broadcasted_iota(jnp.int32, (16,), 0)
    half = lanes >> 1
    feat = lanes & 1
    zeros16 = jnp.zeros((16,), jnp.float32)
    wbase = wid * PW

    pltpu.sync_copy(w_hbm, w_v.at[pl.ds(0, 32)])
    pltpu.sync_copy(res_hbm, res_v.at[pl.ds(0, 16)])

    pltpu.async_copy(xs_hbm.at[pl.ds(wbase, CH)], xs_v.at[pl.ds(0, CH)], psem)
    pltpu.async_copy(ys_hbm.at[pl.ds(wbase, CH)], ys_v.at[pl.ds(0, CH)], psem)
    pltpu.async_copy(zs_hbm.at[pl.ds(wbase, CH)], zs_v.at[pl.ds(0, CH)], psem)

    def zero_body(k, _):
        out_v[pl.ds(k * 16, 16)] = zeros16
        return 0
    lax.fori_loop(0, 2 * OB // 16, zero_body, 0, unroll=8)

    wpair0 = w_v[pl.ds(0, 16)]
    w00 = wpair0[0]
    w01 = wpair0[1]
    active0 = jnp.logical_or(w00 != 0.0, w01 != 0.0)

    @pl.when(active0)
    def _build_lut0():
        tstart = sid * TSHARE

        def lut_idx_body(g, _):
            m = tstart + g * 16 + lanes
            n = m >> 1
            i = (n * 6473) >> 21
            r = n - i * (S0 * S0)
            jj = (r * 3641) >> 16
            kk = r - jj * S0
            h = (i.astype(jnp.uint32)
                 ^ (jj.astype(jnp.uint32) * P1)
                 ^ (kk.astype(jnp.uint32) * P2))
            tidx = (h & jnp.uint32(TABLE_SIZE - 1)).astype(jnp.int32)
            idx2_v[pl.ds(g * 16, 16)] = (
                ((tidx >> 7) << 8) + (feat << 7) + (tidx & 127))
            return 0
        lax.fori_loop(0, NLUTPT, lut_idx_body, 0, unroll=4)

        copies = []
        for s in range(NLUTST):
            copies.append(pltpu.async_copy(
                tbl_hbm.at[idx2_v.at[pl.ds(s * 128, 128)]],
                rows_v.at[pl.ds(s * 128, 128)], sem))
        for cp in copies:
            cp.wait()
        pltpu.sync_copy(rows_v.at[pl.ds(0, TSHARE)],
                        lut_sh.at[pl.ds(sid * TSHARE, TSHARE)])
        plsc.subcore_barrier()
        pltpu.sync_copy(lut_sh, rows_v.at[pl.ds(0, 2 * E0P)])

        def deint_body(g, _):
            p16 = g * 16 + lanes
            lut0_v[pl.ds(g * 16, 16)] = plsc.load_gather(rows_v, [p16 * 2])
            lut1_v[pl.ds(g * 16, 16)] = plsc.load_gather(rows_v, [p16 * 2 + 1])
            return 0
        lax.fori_loop(0, NLUTG, deint_body, 0, unroll=4)

    def chunk_body(ci, _):
        pb = (ci & 1) * CH
        ob = (ci & 1) * OB
        base = wbase + ci * CH

        pltpu.make_async_copy(
            xs_hbm.at[pl.ds(0, 3 * CH)], idx2_v.at[pl.ds(0, 3 * CH)], psem
        ).wait()

        @pl.when(ci + 1 < NCHUNK)
        def _prefetch():
            nb = CH - pb
            pltpu.async_copy(xs_hbm.at[pl.ds(base + CH, CH)],
                             xs_v.at[pl.ds(nb, CH)], psem)
            pltpu.async_copy(ys_hbm.at[pl.ds(base + CH, CH)],
                             ys_v.at[pl.ds(nb, CH)], psem)
            pltpu.async_copy(zs_hbm.at[pl.ds(base + CH, CH)],
                             zs_v.at[pl.ds(nb, CH)], psem)

        @pl.when(ci >= 2)
        def _drain_out():
            pltpu.make_async_copy(
                out_hbm.at[pl.ds(0, OB)], out_v.at[pl.ds(0, OB)], osem
            ).wait()

        @pl.when(active0)
        def _level0():
            res0 = jnp.float32(RES0)

            def l0_body(j, _):
                sl = pl.ds(pb + j * 16, 16)
                x = xs_v[sl] * res0
                y = ys_v[sl] * res0
                z = zs_v[sl] * res0
                xi = x.astype(jnp.int32)
                yi = y.astype(jnp.int32)
                zi = z.astype(jnp.int32)
                fx = x - xi.astype(jnp.float32)
                fy = y - yi.astype(jnp.float32)
                fz = z - zi.astype(jnp.float32)
                gx = (1.0 - fx, fx)
                gyz = {}
                for by in (0, 1):
                    for bz in (0, 1):
                        gyz[(by, bz)] = ((fy if by else 1.0 - fy)
                                         * (fz if bz else 1.0 - fz))
                lbase = xi * (S0 * S0) + yi * S0 + zi
                acc0 = zeros16
                acc1 = zeros16
                for c in range(8):
                    bx, by, bz = c & 1, (c >> 1) & 1, (c >> 2) & 1
                    lidx = lbase + OFF0[c]
                    w = gx[bx] * gyz[(by, bz)]
                    acc0 = acc0 + w * plsc.load_gather(lut0_v, [lidx])
                    acc1 = acc1 + w * plsc.load_gather(lut1_v, [lidx])
                off = ob + (j >> 3) * 1024 + (j & 7) * 16
                out_v[pl.ds(off, 16)] = acc0 * w00
                out_v[pl.ds(off + 128, 16)] = acc1 * w01
                return 0
            lax.fori_loop(0, NGROUP, l0_body, 0, unroll=8)

        def level_body(l, _):
            wpair = w_v[pl.ds(2 * l, 16)]
            w0 = wpair[0]
            w1 = wpair[1]
            active = jnp.logical_or(w0 != 0.0, w1 != 0.0)

            @pl.when(active)
            def _do_level():
                res = res_v[pl.ds(l, 16)][0]
                lbase2 = l * (TABLE_SIZE * 2)
                oscal = ob + ((l >> 2) << 13) + ((l & 3) << 8)

                def hash_body(j, _):
                    sl = pl.ds(pb + j * 16, 16)
                    p16 = j * 16 + lanes
                    x = xs_v[sl] * res
                    y = ys_v[sl] * res
                    z = zs_v[sl] * res
                    xi = x.astype(jnp.int32)
                    yi = y.astype(jnp.int32)
                    zi = z.astype(jnp.int32)
                    fx = x - xi.astype(jnp.float32)
                    fy = y - yi.astype(jnp.float32)
                    fz = z - zi.astype(jnp.float32)
                    gx = (1.0 - fx, fx)
                    gy = (1.0 - fy, fy)
                    gz = (1.0 - fz, fz)
                    xu = xi.astype(jnp.uint32)
                    hx = (xu, xu + jnp.uint32(1))
                    hy0 = yi.astype(jnp.uint32) * P1
                    hy = (hy0, hy0 + P1)
                    hz0 = zi.astype(jnp.uint32) * P2
                    hz = (hz0, hz0 + P2)
                    for c in range(8):
                        bx, by, bz = c & 1, (c >> 1) & 1, (c >> 2) & 1
                        h = hx[bx] ^ hy[by] ^ hz[bz]
                        tidx = (h & jnp.uint32(TABLE_SIZE - 1)).astype(jnp.int32)
                        e0 = lbase2 + ((tidx >> 7) << 8) + (tidx & 127)
                        slot = c * (2 * CH) + 2 * p16
                        plsc.store_scatter(idx2_v, [slot], e0)
                        plsc.store_scatter(idx2_v, [slot + 1], e0 + 128)
                        wgt_v[pl.ds(c * CH + j * 16, 16)] = gx[bx] * gy[by] * gz[bz]
                    return 0
                lax.fori_loop(0, NGROUP, hash_body, 0)

                copies = []
                for c in range(8):
                    for s in range(NSUB):
                        off = c * (2 * CH) + s * 128
                        copies.append(pltpu.async_copy(
                            tbl_hbm.at[idx2_v.at[pl.ds(off, 128)]],
                            rows_v.at[pl.ds(off, 128)], sem))
                for cp in copies:
                    cp.wait()

                wsel = jnp.where(feat == 0, w0, w1)

                def acc_body(g, _):
                    p8 = g * 8 + half
                    acc = zeros16
                    for c in range(8):
                        w = plsc.load_gather(wgt_v, [c * CH + p8])
                        r = rows_v[pl.ds(c * (2 * CH) + g * 16, 16)]
                        acc = acc + w * r
                    oidx = oscal + ((p8 >> 7) << 10) + (feat << 7) + (p8 & 127)
                    plsc.store_scatter(out_v, [oidx], acc * wsel)
                    return 0
                lax.fori_loop(0, NPAIR, acc_body, 0)
            return 0
        lax.fori_loop(1, N_LEVELS, level_body, 0)

        for cb in range(4):
            pltpu.async_copy(
                out_v.at[pl.ds(ob + cb * 8192, 8192)],
                out_hbm.at[pl.ds(cb * (N_POS * 8) + base * 8, 8192)], osem)
        return 0
    lax.fori_loop(0, NCHUNK, chunk_body, 0)

    for _ in range(2):
        pltpu.make_async_copy(
            out_hbm.at[pl.ds(0, OB)], out_v.at[pl.ds(0, OB)], osem
        ).wait()


@functools.partial(
    pl.kernel,
    mesh=plsc.VectorSubcoreMesh(core_axis_name="c", subcore_axis_name="s"),
    out_type=jax.ShapeDtypeStruct((N_POS * 32,), jnp.float32),
    compiler_params=pltpu.CompilerParams(needs_layout_passes=False),
    scratch_types=[
        pltpu.VMEM((2 * CH,), jnp.float32),
        pltpu.VMEM((2 * CH,), jnp.float32),
        pltpu.VMEM((2 * CH,), jnp.float32),
        pltpu.VMEM((8 * 2 * CH,), jnp.int32),
        pltpu.VMEM((8 * CH,), jnp.float32),
        pltpu.VMEM((8 * 2 * CH,), jnp.float32),
        pltpu.VMEM((2 * OB,), jnp.float32),
        pltpu.VMEM((48,), jnp.float32),
        pltpu.VMEM((32,), jnp.float32),
        pltpu.VMEM((E0P,), jnp.float32),
        pltpu.VMEM((E0P,), jnp.float32),
        pltpu.VMEM_SHARED((2 * E0P,), jnp.float32),
        pltpu.SemaphoreType.DMA,
        pltpu.SemaphoreType.DMA,
        pltpu.SemaphoreType.DMA,
    ],
)
def _encode(xs_hbm, ys_hbm, zs_hbm, tbl_hbm, w_hbm, res_hbm, out_hbm, *scratch):
    _sc_body(xs_hbm, ys_hbm, zs_hbm, tbl_hbm, w_hbm, res_hbm, out_hbm, *scratch)


def kernel(positions, table, weights):
    xs, ys, zs = positions[:, 0], positions[:, 1], positions[:, 2]
    tblf = (table.reshape(N_LEVELS, TABLE_SIZE // 128, 128, N_FEAT)
            .swapaxes(2, 3)
            .reshape(N_LEVELS * TABLE_SIZE * N_FEAT))
    wfl = weights.reshape(N_LEVELS * N_FEAT)
    res = jnp.asarray(np.array(RES_LIST, dtype=np.float32))
    out = _encode(xs, ys, zs, tblf, wfl, res)
    return (out.reshape(4, N_POS // 128, 8, 128)
            .transpose(1, 3, 0, 2)
            .reshape(N_POS, N_LEVELS * N_FEAT))


# speedup vs baseline: 1.0065x; 1.0065x over previous
import functools

import jax
import jax.numpy as jnp
import numpy as np
from jax import lax
from jax.experimental import pallas as pl
from jax.experimental.pallas import tpu as pltpu
from jax.experimental.pallas import tpu_sc as plsc

N_POS = 262144
N_LEVELS = 16
N_FEAT = 2
LOG2_T = 19
TABLE_SIZE = 1 << LOG2_T
BASE_RES = 16
PER_LEVEL_SCALE = 1.3819129
P1 = np.uint32(2654435761)
P2 = np.uint32(805459861)

NW = 32
PW = N_POS // NW
CH = 1024
NCHUNK = PW // CH
NGROUP = CH // 16
NPAIR = CH // 8
NSUB = 2 * CH // 128
OB = CH * 32

RES_LIST = [int(np.floor(BASE_RES * (PER_LEVEL_SCALE ** l))) for l in range(N_LEVELS)]

RES0 = RES_LIST[0]
S0 = RES0 + 2
E0 = S0 * S0 * S0
E0P = ((2 * E0 + 2047) // 2048) * 1024
NLUTG = E0P // 16
TSHARE = 2 * E0P // 16
NLUTPT = TSHARE // 16
NLUTST = TSHARE // 128
OFF0 = [bx * S0 * S0 + by * S0 + bz
        for bz, by, bx in [((c >> 2) & 1, (c >> 1) & 1, c & 1) for c in range(8)]]


def _sc_body(xs_hbm, ys_hbm, zs_hbm, tbl_hbm, w_hbm, res_hbm, out_hbm,
             xs_v, ys_v, zs_v, idx2_v, wgt_v, rows_v, out_v, w_v, res_v,
             lut0_v, lut1_v, lut_sh, sem, psem, osem):
    sid = lax.axis_index("s")
    wid = sid * 2 + lax.axis_index("c")
    lanes = lax.broadcasted_iota(jnp.int32, (16,), 0)
    half = lanes >> 1
    feat = lanes & 1
    zeros16 = jnp.zeros((16,), jnp.float32)
    wbase = wid * PW

    pltpu.sync_copy(w_hbm, w_v.at[pl.ds(0, 32)])
    pltpu.sync_copy(res_hbm, res_v.at[pl.ds(0, 16)])

    pltpu.async_copy(xs_hbm.at[pl.ds(wbase, CH)], xs_v.at[pl.ds(0, CH)], psem)
    pltpu.async_copy(ys_hbm.at[pl.ds(wbase, CH)], ys_v.at[pl.ds(0, CH)], psem)
    pltpu.async_copy(zs_hbm.at[pl.ds(wbase, CH)], zs_v.at[pl.ds(0, CH)], psem)

    def zero_body(k, _):
        out_v[pl.ds(k * 16, 16)] = zeros16
        return 0
    lax.fori_loop(0, 2 * OB // 16, zero_body, 0, unroll=8)

    wpair0 = w_v[pl.ds(0, 16)]
    w00 = wpair0[0]
    w01 = wpair0[1]
    active0 = jnp.logical_or(w00 != 0.0, w01 != 0.0)

    @pl.when(active0)
    def _build_lut0():
        tstart = sid * TSHARE

        def lut_idx_body(g, _):
            m = tstart + g * 16 + lanes
            n = m >> 1
            i = (n * 6473) >> 21
            r = n - i * (S0 * S0)
            jj = (r * 3641) >> 16
            kk = r - jj * S0
            h = (i.astype(jnp.uint32)
                 ^ (jj.astype(jnp.uint32) * P1)
                 ^ (kk.astype(jnp.uint32) * P2))
            tidx = (h & jnp.uint32(TABLE_SIZE - 1)).astype(jnp.int32)
            idx2_v[pl.ds(g * 16, 16)] = (
                ((tidx >> 7) << 8) + (feat << 7) + (tidx & 127))
            return 0
        lax.fori_loop(0, NLUTPT, lut_idx_body, 0, unroll=4)

        copies = []
        for s in range(NLUTST):
            copies.append(pltpu.async_copy(
                tbl_hbm.at[idx2_v.at[pl.ds(s * 128, 128)]],
                rows_v.at[pl.ds(s * 128, 128)], sem))
        for cp in copies:
            cp.wait()
        pltpu.sync_copy(rows_v.at[pl.ds(0, TSHARE)],
                        lut_sh.at[pl.ds(sid * TSHARE, TSHARE)])
        plsc.subcore_barrier()
        pltpu.sync_copy(lut_sh, rows_v.at[pl.ds(0, 2 * E0P)])

        def deint_body(g, _):
            p16 = g * 16 + lanes
            lut0_v[pl.ds(g * 16, 16)] = plsc.load_gather(rows_v, [p16 * 2])
            lut1_v[pl.ds(g * 16, 16)] = plsc.load_gather(rows_v, [p16 * 2 + 1])
            return 0
        lax.fori_loop(0, NLUTG, deint_body, 0, unroll=4)

    def chunk_body(ci, _):
        pb = (ci & 1) * CH
        ob = (ci & 1) * OB
        base = wbase + ci * CH

        pltpu.make_async_copy(
            xs_hbm.at[pl.ds(0, 3 * CH)], idx2_v.at[pl.ds(0, 3 * CH)], psem
        ).wait()

        @pl.when(ci + 1 < NCHUNK)
        def _prefetch():
            nb = CH - pb
            pltpu.async_copy(xs_hbm.at[pl.ds(base + CH, CH)],
                             xs_v.at[pl.ds(nb, CH)], psem)
            pltpu.async_copy(ys_hbm.at[pl.ds(base + CH, CH)],
                             ys_v.at[pl.ds(nb, CH)], psem)
            pltpu.async_copy(zs_hbm.at[pl.ds(base + CH, CH)],
                             zs_v.at[pl.ds(nb, CH)], psem)

        @pl.when(ci >= 2)
        def _drain_out():
            pltpu.make_async_copy(
                out_hbm.at[pl.ds(0, OB)], out_v.at[pl.ds(0, OB)], osem
            ).wait()

        @pl.when(active0)
        def _level0():
            res0 = jnp.float32(RES0)

            def l0_body(j, _):
                sl = pl.ds(pb + j * 16, 16)
                x = xs_v[sl] * res0
                y = ys_v[sl] * res0
                z = zs_v[sl] * res0
                xi = x.astype(jnp.int32)
                yi = y.astype(jnp.int32)
                zi = z.astype(jnp.int32)
                fx = x - xi.astype(jnp.float32)
                fy = y - yi.astype(jnp.float32)
                fz = z - zi.astype(jnp.float32)
                gx = (1.0 - fx, fx)
                gyz = {}
                for by in (0, 1):
                    for bz in (0, 1):
                        gyz[(by, bz)] = ((fy if by else 1.0 - fy)
                                         * (fz if bz else 1.0 - fz))
                lbase = xi * (S0 * S0) + yi * S0 + zi
                acc0 = zeros16
                acc1 = zeros16
                for c in range(8):
                    bx, by, bz = c & 1, (c >> 1) & 1, (c >> 2) & 1
                    lidx = lbase + OFF0[c]
                    w = gx[bx] * gyz[(by, bz)]
                    acc0 = acc0 + w * plsc.load_gather(lut0_v, [lidx])
                    acc1 = acc1 + w * plsc.load_gather(lut1_v, [lidx])
                off = ob + (j >> 3) * 1024 + (j & 7) * 16
                out_v[pl.ds(off, 16)] = acc0 * w00
                out_v[pl.ds(off + 128, 16)] = acc1 * w01
                return 0
            lax.fori_loop(0, NGROUP, l0_body, 0, unroll=4)

        def level_body(l, _):
            wpair = w_v[pl.ds(2 * l, 16)]
            w0 = wpair[0]
            w1 = wpair[1]
            active = jnp.logical_or(w0 != 0.0, w1 != 0.0)

            @pl.when(active)
            def _do_level():
                res = res_v[pl.ds(l, 16)][0]
                lbase2 = l * (TABLE_SIZE * 2)
                oscal = ob + ((l >> 2) << 13) + ((l & 3) << 8)

                def hash_body(j, _):
                    sl = pl.ds(pb + j * 16, 16)
                    p16 = j * 16 + lanes
                    x = xs_v[sl] * res
                    y = ys_v[sl] * res
                    z = zs_v[sl] * res
                    xi = x.astype(jnp.int32)
                    yi = y.astype(jnp.int32)
                    zi = z.astype(jnp.int32)
                    fx = x - xi.astype(jnp.float32)
                    fy = y - yi.astype(jnp.float32)
                    fz = z - zi.astype(jnp.float32)
                    gx = (1.0 - fx, fx)
                    gy = (1.0 - fy, fy)
                    gz = (1.0 - fz, fz)
                    xu = xi.astype(jnp.uint32)
                    hx = (xu, xu + jnp.uint32(1))
                    hy0 = yi.astype(jnp.uint32) * P1
                    hy = (hy0, hy0 + P1)
                    hz0 = zi.astype(jnp.uint32) * P2
                    hz = (hz0, hz0 + P2)
                    for c in range(8):
                        bx, by, bz = c & 1, (c >> 1) & 1, (c >> 2) & 1
                        h = hx[bx] ^ hy[by] ^ hz[bz]
                        tidx = (h & jnp.uint32(TABLE_SIZE - 1)).astype(jnp.int32)
                        e0 = lbase2 + ((tidx >> 7) << 8) + (tidx & 127)
                        slot = c * (2 * CH) + 2 * p16
                        plsc.store_scatter(idx2_v, [slot], e0)
                        plsc.store_scatter(idx2_v, [slot + 1], e0 + 128)
                        wgt_v[pl.ds(c * CH + j * 16, 16)] = gx[bx] * gy[by] * gz[bz]
                    return 0
                lax.fori_loop(0, NGROUP, hash_body, 0)

                copies = []
                for c in range(8):
                    for s in range(NSUB):
                        off = c * (2 * CH) + s * 128
                        copies.append(pltpu.async_copy(
                            tbl_hbm.at[idx2_v.at[pl.ds(off, 128)]],
                            rows_v.at[pl.ds(off, 128)], sem))
                for cp in copies:
                    cp.wait()

                wsel = jnp.where(feat == 0, w0, w1)

                def acc_body(g, _):
                    p8 = g * 8 + half
                    acc = zeros16
                    for c in range(8):
                        w = plsc.load_gather(wgt_v, [c * CH + p8])
                        r = rows_v[pl.ds(c * (2 * CH) + g * 16, 16)]
                        acc = acc + w * r
                    oidx = oscal + ((p8 >> 7) << 10) + (feat << 7) + (p8 & 127)
                    plsc.store_scatter(out_v, [oidx], acc * wsel)
                    return 0
                lax.fori_loop(0, NPAIR, acc_body, 0)
            return 0
        lax.fori_loop(1, N_LEVELS, level_body, 0)

        for cb in range(4):
            pltpu.async_copy(
                out_v.at[pl.ds(ob + cb * 8192, 8192)],
                out_hbm.at[pl.ds(cb * (N_POS * 8) + base * 8, 8192)], osem)
        return 0
    lax.fori_loop(0, NCHUNK, chunk_body, 0)

    for _ in range(2):
        pltpu.make_async_copy(
            out_hbm.at[pl.ds(0, OB)], out_v.at[pl.ds(0, OB)], osem
        ).wait()


@functools.partial(
    pl.kernel,
    mesh=plsc.VectorSubcoreMesh(core_axis_name="c", subcore_axis_name="s"),
    out_type=jax.ShapeDtypeStruct((N_POS * 32,), jnp.float32),
    compiler_params=pltpu.CompilerParams(needs_layout_passes=False),
    scratch_types=[
        pltpu.VMEM((2 * CH,), jnp.float32),
        pltpu.VMEM((2 * CH,), jnp.float32),
        pltpu.VMEM((2 * CH,), jnp.float32),
        pltpu.VMEM((8 * 2 * CH,), jnp.int32),
        pltpu.VMEM((8 * CH,), jnp.float32),
        pltpu.VMEM((8 * 2 * CH,), jnp.float32),
        pltpu.VMEM((2 * OB,), jnp.float32),
        pltpu.VMEM((48,), jnp.float32),
        pltpu.VMEM((32,), jnp.float32),
        pltpu.VMEM((E0P,), jnp.float32),
        pltpu.VMEM((E0P,), jnp.float32),
        pltpu.VMEM_SHARED((2 * E0P,), jnp.float32),
        pltpu.SemaphoreType.DMA,
        pltpu.SemaphoreType.DMA,
        pltpu.SemaphoreType.DMA,
    ],
)
def _encode(xs_hbm, ys_hbm, zs_hbm, tbl_hbm, w_hbm, res_hbm, out_hbm, *scratch):
    _sc_body(xs_hbm, ys_hbm, zs_hbm, tbl_hbm, w_hbm, res_hbm, out_hbm, *scratch)


def kernel(positions, table, weights):
    pos_t = positions.T
    xs, ys, zs = pos_t[0], pos_t[1], pos_t[2]
    tblf = (table.reshape(N_LEVELS, TABLE_SIZE // 128, 128, N_FEAT)
            .swapaxes(2, 3)
            .reshape(N_LEVELS * TABLE_SIZE * N_FEAT))
    wfl = weights.reshape(N_LEVELS * N_FEAT)
    res = jnp.asarray(np.array(RES_LIST, dtype=np.float32))
    out = _encode(xs, ys, zs, tblf, wfl, res)
    return (out.reshape(4, N_POS // 128, 8, 128)
            .transpose(1, 3, 0, 2)
            .reshape(N_POS, N_LEVELS * N_FEAT))


# speedup vs baseline: 1.0628x; 1.0559x over previous
import functools

import jax
import jax.numpy as jnp
import numpy as np
from jax import lax
from jax.experimental import pallas as pl
from jax.experimental.pallas import tpu as pltpu
from jax.experimental.pallas import tpu_sc as plsc

N_POS = 262144
N_LEVELS = 16
N_FEAT = 2
LOG2_T = 19
TABLE_SIZE = 1 << LOG2_T
BASE_RES = 16
PER_LEVEL_SCALE = 1.3819129
P1 = np.uint32(2654435761)
P2 = np.uint32(805459861)

NW = 32
PW = N_POS // NW
CH = 1024
NCHUNK = PW // CH
NGROUP = CH // 16
NPAIR = CH // 8
NSUB = 2 * CH // 128
OB = CH * 32

RES_LIST = [int(np.floor(BASE_RES * (PER_LEVEL_SCALE ** l))) for l in range(N_LEVELS)]

RES0 = RES_LIST[0]
S0 = RES0 + 2
E0 = S0 * S0 * S0
E0P = ((2 * E0 + 2047) // 2048) * 1024
NLUTG = E0P // 16
TSHARE = 2 * E0P // 16
NLUTPT = TSHARE // 16
NLUTST = TSHARE // 128
OFF0 = [bx * S0 * S0 + by * S0 + bz
        for bz, by, bx in [((c >> 2) & 1, (c >> 1) & 1, c & 1) for c in range(8)]]


def _sc_body(xs_hbm, ys_hbm, zs_hbm, tbl_hbm, w_hbm, res_hbm, out_hbm,
             xs_v, ys_v, zs_v, idx2_v, wgt_v, rows_v, out_v, w_v, res_v,
             lut0_v, lut1_v, lut_sh, sem, psem, osem):
    sid = lax.axis_index("s")
    wid = sid * 2 + lax.axis_index("c")
    lanes = lax.broadcasted_iota(jnp.int32, (16,), 0)
    half = lanes >> 1
    feat = lanes & 1
    zeros16 = jnp.zeros((16,), jnp.float32)
    wbase = wid * PW

    pltpu.sync_copy(w_hbm, w_v.at[pl.ds(0, 32)])
    pltpu.sync_copy(res_hbm, res_v.at[pl.ds(0, 16)])

    pltpu.async_copy(xs_hbm.at[pl.ds(wbase, CH)], xs_v.at[pl.ds(0, CH)], psem)
    pltpu.async_copy(ys_hbm.at[pl.ds(wbase, CH)], ys_v.at[pl.ds(0, CH)], psem)
    pltpu.async_copy(zs_hbm.at[pl.ds(wbase, CH)], zs_v.at[pl.ds(0, CH)], psem)

    def zero_body(k, _):
        out_v[pl.ds(k * 16, 16)] = zeros16
        return 0
    lax.fori_loop(0, 2 * OB // 16, zero_body, 0, unroll=8)

    wpair0 = w_v[pl.ds(0, 16)]
    w00 = wpair0[0]
    w01 = wpair0[1]
    active0 = jnp.logical_or(w00 != 0.0, w01 != 0.0)

    @pl.when(active0)
    def _build_lut0():
        fscal = jnp.where(sid >= 8, 1, 0)
        nbase = sid * TSHARE - fscal * E0P

        def lut_idx_body(g, _):
            n = nbase + g * 16 + lanes
            i = (n * 6473) >> 21
            r = n - i * (S0 * S0)
            jj = (r * 3641) >> 16
            kk = r - jj * S0
            h = (i.astype(jnp.uint32)
                 ^ (jj.astype(jnp.uint32) * P1)
                 ^ (kk.astype(jnp.uint32) * P2))
            tidx = (h & jnp.uint32(TABLE_SIZE - 1)).astype(jnp.int32)
            idx2_v[pl.ds(g * 16, 16)] = (
                ((tidx >> 7) << 8) + (fscal << 7) + (tidx & 127))
            return 0
        lax.fori_loop(0, NLUTPT, lut_idx_body, 0, unroll=4)

        copies = []
        for s in range(NLUTST):
            copies.append(pltpu.async_copy(
                tbl_hbm.at[idx2_v.at[pl.ds(s * 128, 128)]],
                rows_v.at[pl.ds(s * 128, 128)], sem))
        for cp in copies:
            cp.wait()
        pltpu.sync_copy(rows_v.at[pl.ds(0, TSHARE)],
                        lut_sh.at[pl.ds(sid * TSHARE, TSHARE)])
        plsc.subcore_barrier()
        pltpu.sync_copy(lut_sh.at[pl.ds(0, E0P)], lut0_v)
        pltpu.sync_copy(lut_sh.at[pl.ds(E0P, E0P)], lut1_v)

    def chunk_body(ci, _):
        pb = (ci & 1) * CH
        ob = (ci & 1) * OB
        base = wbase + ci * CH

        pltpu.make_async_copy(
            xs_hbm.at[pl.ds(0, 3 * CH)], idx2_v.at[pl.ds(0, 3 * CH)], psem
        ).wait()

        @pl.when(ci + 1 < NCHUNK)
        def _prefetch():
            nb = CH - pb
            pltpu.async_copy(xs_hbm.at[pl.ds(base + CH, CH)],
                             xs_v.at[pl.ds(nb, CH)], psem)
            pltpu.async_copy(ys_hbm.at[pl.ds(base + CH, CH)],
                             ys_v.at[pl.ds(nb, CH)], psem)
            pltpu.async_copy(zs_hbm.at[pl.ds(base + CH, CH)],
                             zs_v.at[pl.ds(nb, CH)], psem)

        @pl.when(ci >= 2)
        def _drain_out():
            pltpu.make_async_copy(
                out_hbm.at[pl.ds(0, OB)], out_v.at[pl.ds(0, OB)], osem
            ).wait()

        @pl.when(active0)
        def _level0():
            res0 = jnp.float32(RES0)

            def l0_body(j, _):
                sl = pl.ds(pb + j * 16, 16)
                x = xs_v[sl] * res0
                y = ys_v[sl] * res0
                z = zs_v[sl] * res0
                xi = x.astype(jnp.int32)
                yi = y.astype(jnp.int32)
                zi = z.astype(jnp.int32)
                fx = x - xi.astype(jnp.float32)
                fy = y - yi.astype(jnp.float32)
                fz = z - zi.astype(jnp.float32)
                gx = (1.0 - fx, fx)
                gyz = {}
                for by in (0, 1):
                    for bz in (0, 1):
                        gyz[(by, bz)] = ((fy if by else 1.0 - fy)
                                         * (fz if bz else 1.0 - fz))
                lbase = xi * (S0 * S0) + yi * S0 + zi
                acc0 = zeros16
                acc1 = zeros16
                for c in range(8):
                    bx, by, bz = c & 1, (c >> 1) & 1, (c >> 2) & 1
                    lidx = lbase + OFF0[c]
                    w = gx[bx] * gyz[(by, bz)]
                    acc0 = acc0 + w * plsc.load_gather(lut0_v, [lidx])
                    acc1 = acc1 + w * plsc.load_gather(lut1_v, [lidx])
                off = ob + (j >> 3) * 1024 + (j & 7) * 16
                out_v[pl.ds(off, 16)] = acc0 * w00
                out_v[pl.ds(off + 128, 16)] = acc1 * w01
                return 0
            lax.fori_loop(0, NGROUP, l0_body, 0, unroll=4)

        def level_body(l, _):
            wpair = w_v[pl.ds(2 * l, 16)]
            w0 = wpair[0]
            w1 = wpair[1]
            active = jnp.logical_or(w0 != 0.0, w1 != 0.0)

            @pl.when(active)
            def _do_level():
                res = res_v[pl.ds(l, 16)][0]
                lbase2 = l * (TABLE_SIZE * 2)
                oscal = ob + ((l >> 2) << 13) + ((l & 3) << 8)

                def hash_body(j, _):
                    sl = pl.ds(pb + j * 16, 16)
                    p16 = j * 16 + lanes
                    x = xs_v[sl] * res
                    y = ys_v[sl] * res
                    z = zs_v[sl] * res
                    xi = x.astype(jnp.int32)
                    yi = y.astype(jnp.int32)
                    zi = z.astype(jnp.int32)
                    fx = x - xi.astype(jnp.float32)
                    fy = y - yi.astype(jnp.float32)
                    fz = z - zi.astype(jnp.float32)
                    gx = (1.0 - fx, fx)
                    gy = (1.0 - fy, fy)
                    gz = (1.0 - fz, fz)
                    xu = xi.astype(jnp.uint32)
                    hx = (xu, xu + jnp.uint32(1))
                    hy0 = yi.astype(jnp.uint32) * P1
                    hy = (hy0, hy0 + P1)
                    hz0 = zi.astype(jnp.uint32) * P2
                    hz = (hz0, hz0 + P2)
                    for c in range(8):
                        bx, by, bz = c & 1, (c >> 1) & 1, (c >> 2) & 1
                        h = hx[bx] ^ hy[by] ^ hz[bz]
                        tidx = (h & jnp.uint32(TABLE_SIZE - 1)).astype(jnp.int32)
                        e0 = lbase2 + ((tidx >> 7) << 8) + (tidx & 127)
                        slot = c * (2 * CH) + 2 * p16
                        plsc.store_scatter(idx2_v, [slot], e0)
                        plsc.store_scatter(idx2_v, [slot + 1], e0 + 128)
                        wgt_v[pl.ds(c * CH + j * 16, 16)] = gx[bx] * gy[by] * gz[bz]
                    return 0
                lax.fori_loop(0, NGROUP, hash_body, 0)

                copies = []
                for c in range(8):
                    for s in range(NSUB):
                        off = c * (2 * CH) + s * 128
                        copies.append(pltpu.async_copy(
                            tbl_hbm.at[idx2_v.at[pl.ds(off, 128)]],
                            rows_v.at[pl.ds(off, 128)], sem))
                for cp in copies:
                    cp.wait()

                wsel = jnp.where(feat == 0, w0, w1)

                def acc_body(g, _):
                    p8 = g * 8 + half
                    acc = zeros16
                    for c in range(8):
                        w = plsc.load_gather(wgt_v, [c * CH + p8])
                        r = rows_v[pl.ds(c * (2 * CH) + g * 16, 16)]
                        acc = acc + w * r
                    oidx = oscal + ((p8 >> 7) << 10) + (feat << 7) + (p8 & 127)
                    plsc.store_scatter(out_v, [oidx], acc * wsel)
                    return 0
                lax.fori_loop(0, NPAIR, acc_body, 0)
            return 0
        lax.fori_loop(1, N_LEVELS, level_body, 0)

        for cb in range(4):
            pltpu.async_copy(
                out_v.at[pl.ds(ob + cb * 8192, 8192)],
                out_hbm.at[pl.ds(cb * (N_POS * 8) + base * 8, 8192)], osem)
        return 0
    lax.fori_loop(0, NCHUNK, chunk_body, 0)

    for _ in range(2):
        pltpu.make_async_copy(
            out_hbm.at[pl.ds(0, OB)], out_v.at[pl.ds(0, OB)], osem
        ).wait()


@functools.partial(
    pl.kernel,
    mesh=plsc.VectorSubcoreMesh(core_axis_name="c", subcore_axis_name="s"),
    out_type=jax.ShapeDtypeStruct((N_POS * 32,), jnp.float32),
    compiler_params=pltpu.CompilerParams(needs_layout_passes=False),
    scratch_types=[
        pltpu.VMEM((2 * CH,), jnp.float32),
        pltpu.VMEM((2 * CH,), jnp.float32),
        pltpu.VMEM((2 * CH,), jnp.float32),
        pltpu.VMEM((8 * 2 * CH,), jnp.int32),
        pltpu.VMEM((8 * CH,), jnp.float32),
        pltpu.VMEM((8 * 2 * CH,), jnp.float32),
        pltpu.VMEM((2 * OB,), jnp.float32),
        pltpu.VMEM((48,), jnp.float32),
        pltpu.VMEM((32,), jnp.float32),
        pltpu.VMEM((E0P,), jnp.float32),
        pltpu.VMEM((E0P,), jnp.float32),
        pltpu.VMEM_SHARED((2 * E0P,), jnp.float32),
        pltpu.SemaphoreType.DMA,
        pltpu.SemaphoreType.DMA,
        pltpu.SemaphoreType.DMA,
    ],
)
def _encode(xs_hbm, ys_hbm, zs_hbm, tbl_hbm, w_hbm, res_hbm, out_hbm, *scratch):
    _sc_body(xs_hbm, ys_hbm, zs_hbm, tbl_hbm, w_hbm, res_hbm, out_hbm, *scratch)


def kernel(positions, table, weights):
    xs, ys, zs = positions[:, 0], positions[:, 1], positions[:, 2]
    tblf = (table.reshape(N_LEVELS, TABLE_SIZE // 128, 128, N_FEAT)
            .swapaxes(2, 3)
            .reshape(N_LEVELS * TABLE_SIZE * N_FEAT))
    wfl = weights.reshape(N_LEVELS * N_FEAT)
    res = jnp.asarray(np.array(RES_LIST, dtype=np.float32))
    out = _encode(xs, ys, zs, tblf, wfl, res)
    return (out.reshape(4, N_POS // 128, 8, 128)
            .transpose(1, 3, 0, 2)
            .reshape(N_POS, N_LEVELS * N_FEAT))
